# baseline (speedup 1.0000x reference)
import jax
import jax.numpy as jnp
from jax.experimental import pallas as pl
from jax.experimental.pallas import tpu as pltpu

E = 64
H = 1024
MOE_I = 512
SHARED_I = 4096
T = 64
SHARED_BLK = 1024


def _dot_t(a, b):
    return jax.lax.dot_general(
        a.astype(jnp.bfloat16), b.astype(jnp.bfloat16),
        (((1,), (1,)), ((), ())), preferred_element_type=jnp.float32)


def _router_body(x_ref, gw_ref, idx_ref):
    logits = jax.lax.dot_general(
        x_ref[...], gw_ref[...], (((1,), (1,)), ((), ())),
        preferred_element_type=jnp.float32)
    m = jnp.max(logits, axis=1, keepdims=True)
    eiota = jax.lax.broadcasted_iota(jnp.int32, (T, E), 1)
    cand = jnp.where(logits >= m, eiota, E)
    idx_ref[...] = jnp.min(cand, axis=1, keepdims=True)


def _shared_body(x_ref, sg_ref, su_ref, sd_ref, out_ref):
    j = pl.program_id(0)

    @pl.when(j == 0)
    def _():
        out_ref[...] = jnp.zeros_like(out_ref)

    x = x_ref[...]
    g = _dot_t(x, sg_ref[...])
    u = _dot_t(x, su_ref[...])
    act = jax.nn.silu(g) * u
    out_ref[...] += _dot_t(act, sd_ref[...])


def _moe_body(order_ref, n_ref, x_ref, top1_ref, shared_ref,
              wg_ref, wu_ref, wd_ref, out_ref):
    i = pl.program_id(0)

    @pl.when(i == 0)
    def _():
        out_ref[...] = shared_ref[...]

    @pl.when(i < n_ref[0])
    def _():
        e = order_ref[i]
        x = x_ref[...]
        g = _dot_t(x, wg_ref[0])
        u = _dot_t(x, wu_ref[0])
        act = jax.nn.silu(g) * u
        o = _dot_t(act, wd_ref[0])
        mask = (top1_ref[...] == e).astype(jnp.float32)
        out_ref[...] += o * mask


def kernel(hidden_states, gate_w, expert_gate_w, expert_up_w, expert_down_w,
           shared_gate_w, shared_up_w, shared_down_w):
    bsz, seq_len, hidden = hidden_states.shape
    x = hidden_states.reshape(T, H)

    top1 = pl.pallas_call(
        _router_body,
        out_shape=jax.ShapeDtypeStruct((T, 1), jnp.int32),
    )(x, gate_w)

    idx = top1[:, 0]
    active = jnp.zeros((E,), jnp.int32).at[idx].set(1)
    n = jnp.sum(active).astype(jnp.int32)
    order = jnp.argsort(1 - active).astype(jnp.int32)
    last = order[jnp.maximum(n - 1, 0)]
    order = jnp.where(jnp.arange(E, dtype=jnp.int32) < n, order, last)

    shared_out = pl.pallas_call(
        _shared_body,
        grid=(SHARED_I // SHARED_BLK,),
        in_specs=[
            pl.BlockSpec((T, H), lambda j: (0, 0)),
            pl.BlockSpec((SHARED_BLK, H), lambda j: (j, 0)),
            pl.BlockSpec((SHARED_BLK, H), lambda j: (j, 0)),
            pl.BlockSpec((H, SHARED_BLK), lambda j: (0, j)),
        ],
        out_specs=pl.BlockSpec((T, H), lambda j: (0, 0)),
        out_shape=jax.ShapeDtypeStruct((T, H), jnp.float32),
    )(x, shared_gate_w, shared_up_w, shared_down_w)

    out = pl.pallas_call(
        _moe_body,
        grid_spec=pltpu.PrefetchScalarGridSpec(
            num_scalar_prefetch=2,
            grid=(E,),
            in_specs=[
                pl.BlockSpec((T, H), lambda i, order, nn: (0, 0)),
                pl.BlockSpec((T, 1), lambda i, order, nn: (0, 0)),
                pl.BlockSpec((T, H), lambda i, order, nn: (0, 0)),
                pl.BlockSpec((1, MOE_I, H), lambda i, order, nn: (order[i], 0, 0)),
                pl.BlockSpec((1, MOE_I, H), lambda i, order, nn: (order[i], 0, 0)),
                pl.BlockSpec((1, H, MOE_I), lambda i, order, nn: (order[i], 0, 0)),
            ],
            out_specs=pl.BlockSpec((T, H), lambda i, order, nn: (0, 0)),
        ),
        out_shape=jax.ShapeDtypeStruct((T, H), jnp.float32),
    )(order, n.reshape(1), x, top1, shared_out,
      expert_gate_w, expert_up_w, expert_down_w)

    return out.reshape(bsz, seq_len, hidden)


# speedup vs baseline: 1.0033x; 1.0033x over previous
import jax
import jax.numpy as jnp
from jax.experimental import pallas as pl
from jax.experimental.pallas import tpu as pltpu

E = 64
H = 1024
MOE_I = 512
SHARED_I = 4096
T = 64
SBLK = 512
NSH = SHARED_I // SBLK


def _dot_t(a, b):
    return jax.lax.dot_general(a, b, (((1,), (1,)), ((), ())),
                               preferred_element_type=jnp.float32)


def _router_body(x_ref, gw_ref, idx_ref):
    logits = _dot_t(x_ref[...], gw_ref[...])
    m = jnp.max(logits, axis=1, keepdims=True)
    eiota = jax.lax.broadcasted_iota(jnp.int32, (T, E), 1)
    cand = jnp.where(logits >= m, eiota, E)
    idx_ref[...] = jnp.min(cand, axis=1, keepdims=True)


def _fused_body(order_ref, n_ref, x_ref, top1_ref, sg_ref, su_ref, sd_ref,
                wg_ref, wu_ref, wd_ref, out_ref):
    i = pl.program_id(0)

    @pl.when(i == 0)
    def _():
        out_ref[...] = jnp.zeros_like(out_ref)

    @pl.when(i < NSH)
    def _():
        x = x_ref[...]
        g = _dot_t(x, sg_ref[...])
        u = _dot_t(x, su_ref[...])
        act = jax.nn.silu(g) * u
        out_ref[...] += _dot_t(act, sd_ref[...])

    @pl.when((i >= NSH) & (i - NSH < n_ref[0]))
    def _():
        e = order_ref[i - NSH]
        x = x_ref[...]
        g = _dot_t(x, wg_ref[0])
        u = _dot_t(x, wu_ref[0])
        act = jax.nn.silu(g) * u
        o = _dot_t(act, wd_ref[0])
        mask = (top1_ref[...] == e).astype(jnp.float32)
        out_ref[...] += o * mask


def kernel(hidden_states, gate_w, expert_gate_w, expert_up_w, expert_down_w,
           shared_gate_w, shared_up_w, shared_down_w):
    bsz, seq_len, hidden = hidden_states.shape
    x = hidden_states.reshape(T, H)

    top1 = pl.pallas_call(
        _router_body,
        out_shape=jax.ShapeDtypeStruct((T, 1), jnp.int32),
    )(x, gate_w)

    idx = top1[:, 0]
    active = jnp.zeros((E,), jnp.int32).at[idx].set(1)
    n = jnp.sum(active).astype(jnp.int32)
    order = jnp.argsort(1 - active).astype(jnp.int32)
    last = order[jnp.maximum(n - 1, 0)]
    order = jnp.where(jnp.arange(E, dtype=jnp.int32) < n, order, last)

    def _sh(i, order, nn):
        return (jnp.minimum(i, NSH - 1), 0)

    def _sd(i, order, nn):
        return (0, jnp.minimum(i, NSH - 1))

    def _ex(i, order, nn):
        return (order[jnp.maximum(i - NSH, 0)], 0, 0)

    def _exd(i, order, nn):
        return (order[jnp.maximum(i - NSH, 0)], 0, 0)

    out = pl.pallas_call(
        _fused_body,
        grid_spec=pltpu.PrefetchScalarGridSpec(
            num_scalar_prefetch=2,
            grid=(NSH + E,),
            in_specs=[
                pl.BlockSpec((T, H), lambda i, order, nn: (0, 0)),
                pl.BlockSpec((T, 1), lambda i, order, nn: (0, 0)),
                pl.BlockSpec((SBLK, H), _sh),
                pl.BlockSpec((SBLK, H), _sh),
                pl.BlockSpec((H, SBLK), _sd),
                pl.BlockSpec((1, MOE_I, H), _ex),
                pl.BlockSpec((1, MOE_I, H), _ex),
                pl.BlockSpec((1, H, MOE_I), _exd),
            ],
            out_specs=pl.BlockSpec((T, H), lambda i, order, nn: (0, 0)),
        ),
        out_shape=jax.ShapeDtypeStruct((T, H), jnp.float32),
    )(order, n.reshape(1), x, top1, shared_gate_w, shared_up_w, shared_down_w,
      expert_gate_w, expert_up_w, expert_down_w)

    return out.reshape(bsz, seq_len, hidden)
